# baseline (speedup 1.0000x reference)
import jax
import jax.numpy as jnp
from jax.experimental import pallas as pl
from jax.experimental.pallas import tpu as pltpu

B, S, D = 1, 2048, 1024
HD = 64
STD_Q, STD_KV = 12, 3
HSA_H = 4
CHUNK, TOPK = 64, 8
SW, HSW = 512, 512
SCALE = HD ** -0.5
NC = S // CHUNK
QB = 256
NQB = S // QB
NH = STD_Q + STD_KV * 2 + HSA_H + 2
NEG = -1e9
HI = jax.lax.Precision.HIGHEST


def _rot_half(x):
    return jnp.concatenate([-x[:, HD // 2:], x[:, :HD // 2]], axis=1)


def _proj_kernel(x_ref, wq_ref, wk_ref, wv_ref, whq_ref, whk_ref, whv_ref,
                 cos_ref, sin_ref, qn_ref, kn_ref, out_ref):
    x = x_ref[...]
    dn = (((1,), (1,)), ((), ()))
    segs = []
    for w_ref, nh in ((wq_ref, STD_Q), (wk_ref, STD_KV), (wv_ref, STD_KV),
                      (whq_ref, HSA_H), (whk_ref, 1), (whv_ref, 1)):
        y = jax.lax.dot_general(x, w_ref[...], dn,
                                preferred_element_type=jnp.float32)
        segs.extend(y[:, j * HD:(j + 1) * HD] for j in range(nh))
    cos = cos_ref[...]
    sin = sin_ref[...]
    qn = qn_ref[...]
    kn = kn_ref[...]
    for h in range(NH):
        seg = segs[h]
        is_q = h < STD_Q or (18 <= h < 22)
        is_k = (STD_Q <= h < STD_Q + STD_KV) or h == 22
        if is_q or is_k:
            w = qn if is_q else kn
            seg = seg * jax.lax.rsqrt(
                jnp.mean(seg * seg, axis=1, keepdims=True) + 1e-6) * w
            seg = seg * cos + _rot_half(seg) * sin
        out_ref[h, :, :] = seg


def _fused_kernel(proj_ref, e_ref, w3_ref, wo_ref, out_ref, lm_ref,
                  accb_ref, smb_ref):
    qb = pl.program_id(0)
    rows = pl.ds(qb * QB, QB)

    @pl.when(qb == 0)
    def _():
        hk = proj_ref[22]
        lm_ref[...] = jnp.dot(
            e_ref[...], hk, preferred_element_type=jnp.float32,
            precision=HI) * (1.0 / CHUNK)

    hq_cat = jnp.concatenate(
        [proj_ref[18 + h, rows, :] for h in range(HSA_H)], axis=0)
    x = jnp.dot(hq_cat, lm_ref[...].T, preferred_element_type=jnp.float32,
                precision=HI) * SCALE
    i_loc = qb * QB + jax.lax.broadcasted_iota(jnp.int32, (QB, 1), 0)
    off = (i_loc - (HSW - 1)) // CHUNK
    cid = jax.lax.broadcasted_iota(jnp.int32, (QB, NC), 1)
    avail = cid < off
    avail4 = jnp.concatenate([avail] * HSA_H, axis=0)
    xm = jnp.where(avail4, x, NEG)
    v = xm
    rem = jnp.full((HSA_H * QB, 1), float(TOPK), jnp.float32)
    thr = jnp.full((HSA_H * QB, 1), NEG, jnp.float32)
    for _ in range(TOPK):
        cur = jnp.max(v, axis=1, keepdims=True)
        eqm = v == cur
        n = jnp.sum(eqm.astype(jnp.float32), axis=1, keepdims=True)
        thr = jnp.where(rem > 0, cur, thr)
        v = jnp.where(eqm, -jnp.inf, v)
        rem = rem - n
    sel = ((xm >= thr) & avail4).astype(jnp.float32)

    outs = []
    w3 = w3_ref[...]
    v0 = (qb >= 2).astype(jnp.float32)
    v1 = (qb >= 1).astype(jnp.float32)
    cm = jnp.concatenate([jnp.full((1, QB), v0, jnp.float32),
                          jnp.full((1, QB), v1, jnp.float32),
                          jnp.ones((1, QB), jnp.float32)], axis=1)
    kbs = [pl.ds(jnp.maximum(qb - 2 + t, 0) * QB, QB) for t in range(3)]
    for g in range(STD_KV):
        qcat = jnp.concatenate(
            [proj_ref[4 * g + hh, rows, :] for hh in range(4)], axis=0)
        kband = jnp.concatenate(
            [proj_ref[STD_Q + g, kb, :] for kb in kbs], axis=0)
        vband = jnp.concatenate(
            [proj_ref[STD_Q + STD_KV + g, kb, :] for kb in kbs], axis=0)
        lg = jnp.dot(qcat, kband.T,
                     preferred_element_type=jnp.float32) * SCALE
        m = jnp.max(lg, axis=1, keepdims=True)
        p = jnp.exp(lg - m) * w3 * cm
        s = jnp.sum(p, axis=1, keepdims=True)
        o = jnp.dot(p, vband, preferred_element_type=jnp.float32) / s
        outs.extend(o[hh * QB:(hh + 1) * QB] for hh in range(4))

    kf = proj_ref[22]
    vf = proj_ref[23]
    jg = jax.lax.broadcasted_iota(jnp.int32, (QB, S), 1)
    ig = qb * QB + jax.lax.broadcasted_iota(jnp.int32, (QB, S), 0)
    winblk = ((jg <= ig) & (ig - jg < HSW)).astype(jnp.float32)
    HS = S // 2
    for h in range(HSA_H):
        qh = hq_cat[h * QB:(h + 1) * QB]
        tok = jnp.dot(sel[h * QB:(h + 1) * QB], e_ref[...],
                      preferred_element_type=jnp.float32)
        msk = jnp.maximum(winblk, tok)
        lga = jnp.dot(qh, kf[:HS].T, preferred_element_type=jnp.float32) * SCALE
        ma = jnp.max(lga, axis=1, keepdims=True)
        pa = jnp.exp(lga - ma) * msk[:, :HS]
        sa = jnp.sum(pa, axis=1, keepdims=True)
        acca = jnp.dot(pa, vf[:HS], preferred_element_type=jnp.float32)

        @pl.when(qb * QB >= HS)
        def _():
            lgb = jnp.dot(qh, kf[HS:].T,
                          preferred_element_type=jnp.float32) * SCALE
            mb = jnp.max(lgb, axis=1, keepdims=True)
            pb = jnp.exp(lgb - mb) * msk[:, HS:]
            accb_ref[...] = jnp.dot(pb, vf[HS:],
                                    preferred_element_type=jnp.float32)
            smb_ref[:, 0:1] = jnp.sum(pb, axis=1, keepdims=True)
            smb_ref[:, 1:2] = mb

        @pl.when(qb * QB < HS)
        def _():
            accb_ref[...] = jnp.zeros((QB, HD), jnp.float32)
            smb_ref[:, 0:1] = jnp.zeros((QB, 1), jnp.float32)
            smb_ref[:, 1:2] = jnp.full((QB, 1), NEG, jnp.float32)

        sb = smb_ref[:, 0:1]
        mb = smb_ref[:, 1:2]
        accb = accb_ref[...]
        m = jnp.maximum(ma, mb)
        ca = jnp.exp(ma - m)
        cb = jnp.exp(mb - m)
        outs.append((acca * ca + accb * cb) / (sa * ca + sb * cb))

    xcat = jnp.concatenate(outs, axis=1)
    out_ref[...] = jax.lax.dot_general(
        xcat, wo_ref[...], (((1,), (1,)), ((), ())),
        preferred_element_type=jnp.float32)


def kernel(hidden_states, Wq, Wk, Wv, Whq, Whk, Whv, Wo, q_norm_w, k_norm_w):
    x = hidden_states.reshape(S, D)

    pos = jnp.arange(S)
    inv = 1.0 / (10000.0 ** (jnp.arange(0, HD, 2).astype(jnp.float32) / HD))
    ang = pos[:, None] * inv[None, :]
    emb = jnp.concatenate([ang, ang], axis=-1)
    cos = jnp.cos(emb).astype(jnp.float32)
    sin = jnp.sin(emb).astype(jnp.float32)
    qn = q_norm_w.reshape(1, HD)
    kn = k_norm_w.reshape(1, HD)

    proj = pl.pallas_call(
        _proj_kernel,
        grid=(NQB,),
        in_specs=[
            pl.BlockSpec((QB, D), lambda i: (i, 0)),
            pl.BlockSpec((STD_Q * HD, D), lambda i: (0, 0)),
            pl.BlockSpec((STD_KV * HD, D), lambda i: (0, 0)),
            pl.BlockSpec((STD_KV * HD, D), lambda i: (0, 0)),
            pl.BlockSpec((HSA_H * HD, D), lambda i: (0, 0)),
            pl.BlockSpec((HD, D), lambda i: (0, 0)),
            pl.BlockSpec((HD, D), lambda i: (0, 0)),
            pl.BlockSpec((QB, HD), lambda i: (i, 0)),
            pl.BlockSpec((QB, HD), lambda i: (i, 0)),
            pl.BlockSpec((1, HD), lambda i: (0, 0)),
            pl.BlockSpec((1, HD), lambda i: (0, 0)),
        ],
        out_specs=pl.BlockSpec((NH, QB, HD), lambda i: (0, i, 0)),
        out_shape=jax.ShapeDtypeStruct((NH, S, HD), jnp.float32),
    )(x, Wq, Wk, Wv, Whq, Whk, Whv, cos, sin, qn, kn)

    jj = jnp.arange(S)
    cidx = jnp.arange(NC)
    E = (jj[None, :] // CHUNK == cidx[:, None]).astype(jnp.float32)
    r = jnp.arange(QB)
    col = jnp.arange(3 * QB)
    w3 = ((col[None, :] > r[:, None]) &
          (col[None, :] <= r[:, None] + SW)).astype(jnp.float32)
    w3x4 = jnp.tile(w3, (HSA_H, 1))

    out = pl.pallas_call(
        _fused_kernel,
        grid=(NQB,),
        in_specs=[
            pl.BlockSpec((NH, S, HD), lambda i: (0, 0, 0)),
            pl.BlockSpec((NC, S), lambda i: (0, 0)),
            pl.BlockSpec((4 * QB, 3 * QB), lambda i: (0, 0)),
            pl.BlockSpec((D, D), lambda i: (0, 0)),
        ],
        out_specs=pl.BlockSpec((QB, D), lambda i: (i, 0)),
        out_shape=jax.ShapeDtypeStruct((S, D), jnp.float32),
        scratch_shapes=[pltpu.VMEM((NC, HD), jnp.float32),
                        pltpu.VMEM((QB, HD), jnp.float32),
                        pltpu.VMEM((QB, 128), jnp.float32)],
    )(proj, E, w3x4, Wo)

    return out.reshape(B, S, D)


# speedup vs baseline: 1.1213x; 1.1213x over previous
import jax
import jax.numpy as jnp
from jax.experimental import pallas as pl
from jax.experimental.pallas import tpu as pltpu

B, S, D = 1, 2048, 1024
HD = 64
STD_Q, STD_KV = 12, 3
HSA_H = 4
CHUNK, TOPK = 64, 8
SW, HSW = 512, 512
SCALE = HD ** -0.5
NC = S // CHUNK
QB = 256
NQB = S // QB
NH = STD_Q + STD_KV * 2 + HSA_H + 2
NEG = -1e9
HI = jax.lax.Precision.HIGHEST


def _rot_half(x):
    return jnp.concatenate([-x[:, HD // 2:], x[:, :HD // 2]], axis=1)


def _proj_kernel(x_ref, wq_ref, wk_ref, wv_ref, whq_ref, whk_ref, whv_ref,
                 cos_ref, sin_ref, qn_ref, kn_ref, out_ref):
    x = x_ref[...]
    dn = (((1,), (1,)), ((), ()))
    segs = []
    for w_ref, nh in ((wq_ref, STD_Q), (wk_ref, STD_KV), (wv_ref, STD_KV),
                      (whq_ref, HSA_H), (whk_ref, 1), (whv_ref, 1)):
        y = jax.lax.dot_general(x, w_ref[...], dn,
                                preferred_element_type=jnp.float32)
        segs.extend(y[:, j * HD:(j + 1) * HD] for j in range(nh))
    cos = cos_ref[...]
    sin = sin_ref[...]
    qn = qn_ref[...]
    kn = kn_ref[...]
    for h in range(NH):
        seg = segs[h]
        is_q = h < STD_Q or (18 <= h < 22)
        is_k = (STD_Q <= h < STD_Q + STD_KV) or h == 22
        if is_q or is_k:
            w = qn if is_q else kn
            seg = seg * jax.lax.rsqrt(
                jnp.mean(seg * seg, axis=1, keepdims=True) + 1e-6) * w
            seg = seg * cos + _rot_half(seg) * sin
        out_ref[h, :, :] = seg


def _fused_kernel(proj_ref, e_ref, w3_ref, wo_ref, out_ref, lm_ref):
    qb = pl.program_id(0)
    rows = pl.ds(qb * QB, QB)

    @pl.when(qb == 0)
    def _():
        hk = proj_ref[22]
        lm_ref[...] = jnp.dot(
            e_ref[...], hk, preferred_element_type=jnp.float32,
            precision=HI) * (1.0 / CHUNK)

    hq_cat = jnp.concatenate(
        [proj_ref[18 + h, rows, :] for h in range(HSA_H)], axis=0)
    x = jnp.dot(hq_cat, lm_ref[...].T, preferred_element_type=jnp.float32,
                precision=HI) * SCALE
    i_loc = qb * QB + jax.lax.broadcasted_iota(jnp.int32, (QB, 1), 0)
    off = (i_loc - (HSW - 1)) // CHUNK
    cid = jax.lax.broadcasted_iota(jnp.int32, (QB, NC), 1)
    avail = cid < off
    avail4 = jnp.concatenate([avail] * HSA_H, axis=0)
    xm = jnp.where(avail4, x, NEG)
    v = xm
    rem = jnp.full((HSA_H * QB, 1), float(TOPK), jnp.float32)
    thr = jnp.full((HSA_H * QB, 1), NEG, jnp.float32)
    for _ in range(TOPK):
        cur = jnp.max(v, axis=1, keepdims=True)
        eqm = v == cur
        n = jnp.sum(eqm.astype(jnp.float32), axis=1, keepdims=True)
        thr = jnp.where(rem > 0, cur, thr)
        v = jnp.where(eqm, -jnp.inf, v)
        rem = rem - n
    sel = ((xm >= thr) & avail4).astype(jnp.float32)

    outs = []
    w3 = w3_ref[...]
    v0 = (qb >= 2).astype(jnp.float32)
    v1 = (qb >= 1).astype(jnp.float32)
    cm = jnp.concatenate([jnp.full((1, QB), v0, jnp.float32),
                          jnp.full((1, QB), v1, jnp.float32),
                          jnp.ones((1, QB), jnp.float32)], axis=1)
    kbs = [pl.ds(jnp.maximum(qb - 2 + t, 0) * QB, QB) for t in range(3)]
    for g in range(STD_KV):
        qcat = jnp.concatenate(
            [proj_ref[4 * g + hh, rows, :] for hh in range(4)], axis=0)
        kband = jnp.concatenate(
            [proj_ref[STD_Q + g, kb, :] for kb in kbs], axis=0)
        vband = jnp.concatenate(
            [proj_ref[STD_Q + STD_KV + g, kb, :] for kb in kbs], axis=0)
        lg = jnp.dot(qcat, kband.T,
                     preferred_element_type=jnp.float32) * SCALE
        m = jnp.max(lg, axis=1, keepdims=True)
        p = jnp.exp(lg - m) * w3 * cm
        s = jnp.sum(p, axis=1, keepdims=True)
        o = jnp.dot(p, vband, preferred_element_type=jnp.float32) / s
        outs.extend(o[hh * QB:(hh + 1) * QB] for hh in range(4))

    kf = proj_ref[22]
    vf = proj_ref[23]
    jg = jax.lax.broadcasted_iota(jnp.int32, (QB, S), 1)
    ig = qb * QB + jax.lax.broadcasted_iota(jnp.int32, (QB, S), 0)
    winblk = ((jg <= ig) & (ig - jg < HSW)).astype(jnp.float32)
    for h in range(HSA_H):
        qh = hq_cat[h * QB:(h + 1) * QB]
        lg = jnp.dot(qh, kf.T, preferred_element_type=jnp.float32) * SCALE
        tok = jnp.dot(sel[h * QB:(h + 1) * QB], e_ref[...],
                      preferred_element_type=jnp.float32)
        msk = jnp.maximum(winblk, tok)
        m = jnp.max(lg, axis=1, keepdims=True)
        p = jnp.exp(lg - m) * msk
        s = jnp.sum(p, axis=1, keepdims=True)
        outs.append(jnp.dot(p, vf, preferred_element_type=jnp.float32) / s)

    xcat = jnp.concatenate(outs, axis=1)
    out_ref[...] = jax.lax.dot_general(
        xcat, wo_ref[...], (((1,), (1,)), ((), ())),
        preferred_element_type=jnp.float32)


def kernel(hidden_states, Wq, Wk, Wv, Whq, Whk, Whv, Wo, q_norm_w, k_norm_w):
    x = hidden_states.reshape(S, D)

    pos = jnp.arange(S)
    inv = 1.0 / (10000.0 ** (jnp.arange(0, HD, 2).astype(jnp.float32) / HD))
    ang = pos[:, None] * inv[None, :]
    emb = jnp.concatenate([ang, ang], axis=-1)
    cos = jnp.cos(emb).astype(jnp.float32)
    sin = jnp.sin(emb).astype(jnp.float32)
    qn = q_norm_w.reshape(1, HD)
    kn = k_norm_w.reshape(1, HD)

    proj = pl.pallas_call(
        _proj_kernel,
        grid=(NQB,),
        in_specs=[
            pl.BlockSpec((QB, D), lambda i: (i, 0)),
            pl.BlockSpec((STD_Q * HD, D), lambda i: (0, 0)),
            pl.BlockSpec((STD_KV * HD, D), lambda i: (0, 0)),
            pl.BlockSpec((STD_KV * HD, D), lambda i: (0, 0)),
            pl.BlockSpec((HSA_H * HD, D), lambda i: (0, 0)),
            pl.BlockSpec((HD, D), lambda i: (0, 0)),
            pl.BlockSpec((HD, D), lambda i: (0, 0)),
            pl.BlockSpec((QB, HD), lambda i: (i, 0)),
            pl.BlockSpec((QB, HD), lambda i: (i, 0)),
            pl.BlockSpec((1, HD), lambda i: (0, 0)),
            pl.BlockSpec((1, HD), lambda i: (0, 0)),
        ],
        out_specs=pl.BlockSpec((NH, QB, HD), lambda i: (0, i, 0)),
        out_shape=jax.ShapeDtypeStruct((NH, S, HD), jnp.float32),
    )(x, Wq, Wk, Wv, Whq, Whk, Whv, cos, sin, qn, kn)

    jj = jnp.arange(S)
    cidx = jnp.arange(NC)
    E = (jj[None, :] // CHUNK == cidx[:, None]).astype(jnp.float32)
    r = jnp.arange(QB)
    col = jnp.arange(3 * QB)
    w3 = ((col[None, :] > r[:, None]) &
          (col[None, :] <= r[:, None] + SW)).astype(jnp.float32)
    w3x4 = jnp.tile(w3, (HSA_H, 1))

    out = pl.pallas_call(
        _fused_kernel,
        grid=(NQB,),
        in_specs=[
            pl.BlockSpec((NH, S, HD), lambda i: (0, 0, 0)),
            pl.BlockSpec((NC, S), lambda i: (0, 0)),
            pl.BlockSpec((4 * QB, 3 * QB), lambda i: (0, 0)),
            pl.BlockSpec((D, D), lambda i: (0, 0)),
        ],
        out_specs=pl.BlockSpec((QB, D), lambda i: (i, 0)),
        out_shape=jax.ShapeDtypeStruct((S, D), jnp.float32),
        scratch_shapes=[pltpu.VMEM((NC, HD), jnp.float32)],
    )(proj, E, w3x4, Wo)

    return out.reshape(B, S, D)
